# baseline (speedup 1.0000x reference)
import functools
import math

import jax
import jax.numpy as jnp
from jax import lax
from jax.experimental import pallas as pl
from jax.experimental.pallas import tpu as pltpu
from jax.experimental.pallas import tpu_sc as plsc

D_MODEL = 64
SCALE = math.sqrt(D_MODEL)

NC = 2
NS = 16
NW = NC * NS
LANES = 16

CHUNK = 1024
GATHER_ROWS = 128
G = CHUNK // GATHER_ROWS


def _make_gather(B: int):
  b_per_w = B // NW
  n_chunks = b_per_w // CHUNK
  mesh = plsc.VectorSubcoreMesh(core_axis_name="c", subcore_axis_name="s")

  @functools.partial(
      pl.kernel,
      mesh=mesh,
      out_type=jax.ShapeDtypeStruct((B, D_MODEL), jnp.float32),
      scratch_types=[
          pltpu.VMEM((b_per_w,), jnp.int32),
          pltpu.VMEM((CHUNK, D_MODEL), jnp.float32),
          pltpu.SemaphoreType.DMA,
      ],
      compiler_params=pltpu.CompilerParams(use_tc_tiling_on_sc=False),
  )
  def kern(idx_hbm, table_hbm, out_hbm, idx_v, rows_v, sem):
    wid = lax.axis_index("s") * NC + lax.axis_index("c")
    base = wid * b_per_w
    pltpu.sync_copy(idx_hbm.at[pl.ds(base, b_per_w)], idx_v)

    def chunk_body(k, carry):
      off = k * CHUNK
      copies = []
      for j in range(G):
        cp = pltpu.make_async_copy(
            table_hbm.at[idx_v.at[pl.ds(off + j * GATHER_ROWS, GATHER_ROWS)]],
            rows_v.at[pl.ds(j * GATHER_ROWS, GATHER_ROWS)],
            sem,
        )
        cp.start()
        copies.append(cp)
      for cp in copies:
        cp.wait()

      def row_body(r, c):
        for j in range(D_MODEL // LANES):
          sl = pl.ds(j * LANES, LANES)
          rows_v[r, sl] = rows_v[r, sl] * SCALE
        return c

      lax.fori_loop(0, CHUNK, row_body, 0, unroll=2)
      pltpu.sync_copy(rows_v, out_hbm.at[pl.ds(base + off, CHUNK)])
      return carry

    lax.fori_loop(0, n_chunks, chunk_body, 0)

  return kern


def kernel(x, table):
  orig_shape = x.shape
  idx = x.reshape(-1).astype(jnp.int32)
  out = _make_gather(idx.shape[0])(idx, table)
  return out.reshape(orig_shape + (D_MODEL,))


# speedup vs baseline: 1.0526x; 1.0526x over previous
import functools
import math

import jax
import jax.numpy as jnp
from jax import lax
from jax.experimental import pallas as pl
from jax.experimental.pallas import tpu as pltpu
from jax.experimental.pallas import tpu_sc as plsc

D_MODEL = 64
SCALE = math.sqrt(D_MODEL)

NC = 2
NS = 16
NW = NC * NS
LANES = 16

CHUNK = 512
GATHER_ROWS = 128
G = CHUNK // GATHER_ROWS
NBUF = 2


def _make_gather(B: int):
  b_per_w = B // NW
  n_chunks = b_per_w // CHUNK
  assert n_chunks % NBUF == 0
  mesh = plsc.VectorSubcoreMesh(core_axis_name="c", subcore_axis_name="s")

  @functools.partial(
      pl.kernel,
      mesh=mesh,
      out_type=jax.ShapeDtypeStruct((B, D_MODEL), jnp.float32),
      scratch_types=[
          pltpu.VMEM((b_per_w,), jnp.int32),
          pltpu.VMEM((NBUF, CHUNK, D_MODEL), jnp.float32),
          [pltpu.SemaphoreType.DMA] * NBUF,
          [pltpu.SemaphoreType.DMA] * NBUF,
      ],
      compiler_params=pltpu.CompilerParams(use_tc_tiling_on_sc=False),
  )
  def kern(idx_hbm, table_hbm, out_hbm, idx_v, rows_v, gsems, osems):
    wid = lax.axis_index("s") * NC + lax.axis_index("c")
    base = wid * b_per_w
    pltpu.sync_copy(idx_hbm.at[pl.ds(base, b_per_w)], idx_v)

    def fire_gather(k, b):
      for j in range(G):
        pltpu.make_async_copy(
            table_hbm.at[idx_v.at[pl.ds(k * CHUNK + j * GATHER_ROWS,
                                        GATHER_ROWS)]],
            rows_v.at[b].at[pl.ds(j * GATHER_ROWS, GATHER_ROWS)],
            gsems[b],
        ).start()

    def wait_gather(b):
      for j in range(G):
        pltpu.make_async_copy(
            table_hbm.at[idx_v.at[pl.ds(j * GATHER_ROWS, GATHER_ROWS)]],
            rows_v.at[b].at[pl.ds(j * GATHER_ROWS, GATHER_ROWS)],
            gsems[b],
        ).wait()

    def fire_scatter(k, b):
      pltpu.make_async_copy(
          rows_v.at[b], out_hbm.at[pl.ds(base + k * CHUNK, CHUNK)], osems[b]
      ).start()

    def wait_scatter(b):
      pltpu.make_async_copy(
          rows_v.at[b], out_hbm.at[pl.ds(base, CHUNK)], osems[b]
      ).wait()

    fire_gather(0, 0)

    def outer(g0, carry):
      for b in range(NBUF):
        k = g0 + b
        nb = 1 - b
        wait_gather(b)

        @pl.when(k + 1 < n_chunks)
        def _():
          @pl.when(k > 0)
          def _():
            wait_scatter(nb)
          fire_gather(k + 1, nb)

        def row_body(r, c):
          for j in range(D_MODEL // LANES):
            sl = pl.ds(j * LANES, LANES)
            rows_v[b, r, sl] = rows_v[b, r, sl] * SCALE
          return c

        lax.fori_loop(0, CHUNK, row_body, 0, unroll=4)
        fire_scatter(k, b)
      return carry

    lax.fori_loop(0, n_chunks // NBUF, lambda i, c: outer(i * NBUF, c), 0)
    wait_scatter(0)
    wait_scatter(1)

  return kern


def kernel(x, table):
  orig_shape = x.shape
  idx = x.reshape(-1).astype(jnp.int32)
  out = _make_gather(idx.shape[0])(idx, table)
  return out.reshape(orig_shape + (D_MODEL,))


# speedup vs baseline: 1.1395x; 1.0825x over previous
import functools
import math

import jax
import jax.numpy as jnp
from jax import lax
from jax.experimental import pallas as pl
from jax.experimental.pallas import tpu as pltpu
from jax.experimental.pallas import tpu_sc as plsc

D_MODEL = 64
SCALE = math.sqrt(D_MODEL)

NC = 2
NS = 16
NW = NC * NS
LANES = 16

BLK = 128
NBUF = 2


def _make_kernel(B: int, S: int):
  bi_blocks = B // BLK
  bi_per_w = bi_blocks // NW
  n_chunks = S * bi_per_w
  idx_per_w = S * bi_per_w * BLK
  mesh = plsc.VectorSubcoreMesh(core_axis_name="c", subcore_axis_name="s")

  @functools.partial(
      pl.kernel,
      mesh=mesh,
      out_type=jax.ShapeDtypeStruct(
          (S, D_MODEL // 8, bi_blocks, 8, BLK), jnp.float32),
      scratch_types=[
          pltpu.VMEM((idx_per_w,), jnp.int32),
          pltpu.VMEM((NBUF, BLK, D_MODEL), jnp.float32),
          pltpu.VMEM((NBUF, D_MODEL, BLK), jnp.float32),
          [pltpu.SemaphoreType.DMA] * NBUF,
          [pltpu.SemaphoreType.DMA] * NBUF,
          pltpu.SemaphoreType.DMA,
      ],
      compiler_params=pltpu.CompilerParams(use_tc_tiling_on_sc=False, needs_layout_passes=False),
  )
  def kern(idx_hbm, table_hbm, out_hbm, idx_v, rows_v, tbuf_v,
           gsems, osems, isem):
    wid = lax.axis_index("s") * NC + lax.axis_index("c")

    icopies = []
    for j in range(S):
      cp = pltpu.make_async_copy(
          idx_hbm.at[pl.ds(j * B + wid * (bi_per_w * BLK), bi_per_w * BLK)],
          idx_v.at[pl.ds(j * (bi_per_w * BLK), bi_per_w * BLK)],
          isem,
      )
      cp.start()
      icopies.append(cp)
    for cp in icopies:
      cp.wait()

    lane = lax.iota(jnp.int32, LANES)
    perms = [(lane + c) & (LANES - 1) for c in range(LANES)]
    dcols = [lane + db * LANES for db in range(D_MODEL // LANES)]

    def fire_gather(k, b):
      pltpu.make_async_copy(
          table_hbm.at[idx_v.at[pl.ds(k * BLK, BLK)]],
          rows_v.at[b],
          gsems[b],
      ).start()

    def wait_gather(b):
      pltpu.make_async_copy(
          table_hbm.at[idx_v.at[pl.ds(0, BLK)]],
          rows_v.at[b],
          gsems[b],
      ).wait()

    def fire_scatter(k, b):
      j = k >> 2
      bi = wid * bi_per_w + (k & 3)
      for bd in range(D_MODEL // 8):
        pltpu.make_async_copy(
            tbuf_v.at[b].at[pl.ds(bd * 8, 8)],
            out_hbm.at[j, bd, bi],
            osems[b],
        ).start()

    def wait_scatter(b):
      for bd in range(D_MODEL // 8):
        pltpu.make_async_copy(
            tbuf_v.at[b].at[pl.ds(bd * 8, 8)],
            out_hbm.at[0, bd, 0],
            osems[b],
        ).wait()

    fire_gather(0, 0)

    def chunk(k, b, nb):
      wait_gather(b)

      @pl.when(k + 1 < n_chunks)
      def _():
        @pl.when(k > 0)
        def _():
          wait_scatter(nb)
        fire_gather(k + 1, nb)

      def ibody(ib, c):
        rowvs = [perms[cc] + ib * LANES for cc in range(LANES)]
        for db in range(D_MODEL // LANES):
          for cc in range(LANES):
            vec = plsc.load_gather(rows_v.at[b], [rowvs[cc], dcols[db]])
            plsc.store_scatter(tbuf_v.at[b], [dcols[db], rowvs[cc]],
                               vec * SCALE)
        return c

      lax.fori_loop(0, BLK // LANES, ibody, 0)
      fire_scatter(k, b)

    def outer(g0, carry):
      for b in range(NBUF):
        chunk(g0 * NBUF + b, b, 1 - b)
      return carry

    lax.fori_loop(0, n_chunks // NBUF, outer, 0)
    wait_scatter(0)
    wait_scatter(1)

  return kern


def kernel(x, table):
  B, S = x.shape
  idxT = x.T.reshape(-1).astype(jnp.int32)
  out5 = _make_kernel(B, S)(idxT, table)
  return out5.transpose(2, 4, 0, 1, 3).reshape(B, S, D_MODEL)
